# Initial kernel scaffold; baseline (speedup 1.0000x reference)
import jax
import jax.numpy as jnp
from jax.experimental import pallas as pl


def kernel(x, fc1_w, fc1_b, edge_w, edge_b, fc2_w, fc2_b):
    raise NotImplementedError("write your pallas kernel here")


# speedup vs baseline: 512.8988x; 512.8988x over previous
import functools

import jax
import jax.numpy as jnp
from jax import lax
from jax.experimental import pallas as pl
from jax.experimental.pallas import tpu as pltpu
from jax.experimental.pallas import tpu_sc as plsc

_B, _C, _H, _W = 2, 96, 56, 56
_N = _H * _W
_K = 9
_COUT = 2 * _C
_RB = 392
_NBLK = _N // _RB
_CHUNK = 112
_CP = 256
_EPS = 1e-5


def _pre_body(x_ref, w1_ref, b1_ref, w1m2_ref, w2_ref, be_ref,
              xn_ref, a_ref, bb_ref):
    xb = x_ref[0]
    h = lax.dot_general(xb, w1_ref[...], (((0,), (1,)), ((), ())),
                        preferred_element_type=jnp.float32)
    h = h + b1_ref[...]
    mu = jnp.mean(h, axis=0, keepdims=True)
    var = jnp.mean((h - mu) ** 2, axis=0, keepdims=True)
    h = (h - mu) * lax.rsqrt(var + _EPS)
    nrm = jnp.sqrt(jnp.sum(h * h, axis=1, keepdims=True))
    xn_ref[0] = h / jnp.maximum(nrm, 1e-12)
    a_ref[0] = lax.dot_general(h, w1m2_ref[...], (((1,), (1,)), ((), ())),
                               preferred_element_type=jnp.float32) + be_ref[...]
    bbv = lax.dot_general(h, w2_ref[...], (((1,), (1,)), ((), ())),
                          preferred_element_type=jnp.float32)
    bb_ref[0] = jnp.concatenate(
        [bbv, jnp.zeros((_N, _CP - _COUT), jnp.float32)], axis=1)


def _knn_body(xb_ref, xa_ref, idx_ref):
    b = pl.program_id(0)
    xb = xb_ref[0]
    xa = xa_ref[0]
    d = lax.dot_general(xb, xa, (((1,), (1,)), ((), ())),
                        preferred_element_type=jnp.float32)
    cols = lax.broadcasted_iota(jnp.int32, (_RB, _N), 1)
    picks = []
    for _ in range(_K):
        v = jnp.max(d, axis=1, keepdims=True)
        cand = jnp.where(d == v, cols, _N)
        ik = jnp.min(cand, axis=1, keepdims=True)
        picks.append(ik + b * _N)
        d = jnp.where(cols == ik, -jnp.inf, d)
    idx_ref[0] = jnp.concatenate(picks, axis=1)


def _post_body(a_ref, m_ref, w2_ref, b2_ref, x_ref, o_ref):
    y = jnp.maximum(a_ref[0] + m_ref[0][:, :_COUT], 0.0)
    z = lax.dot_general(w2_ref[...], y, (((1,), (1,)), ((), ())),
                        preferred_element_type=jnp.float32)
    z = z + b2_ref[...]
    mu = jnp.mean(z, axis=1, keepdims=True)
    var = jnp.mean((z - mu) ** 2, axis=1, keepdims=True)
    o_ref[0] = (z - mu) * lax.rsqrt(var + _EPS) + x_ref[0]


@functools.lru_cache(maxsize=1)
def _make_gather_max():
    info = plsc.get_sparse_core_info()
    nc, ns = info.num_cores, info.num_subcores
    nw = nc * ns
    tot_chunks = (_B * _N) // _CHUNK
    iters = -(-tot_chunks // nw)
    chunks_per_b = _N // _CHUNK
    nvec = _CP // 16

    mesh = plsc.VectorSubcoreMesh(core_axis_name="c", subcore_axis_name="s")

    @functools.partial(
        pl.kernel, mesh=mesh,
        out_type=jax.ShapeDtypeStruct((_B * _N, _CP), jnp.float32),
        scratch_types=[
            pltpu.VMEM((_CHUNK,), jnp.int32),
            pltpu.VMEM((_CHUNK, _CP), jnp.float32),
            pltpu.VMEM((_CHUNK, _CP), jnp.float32),
            pltpu.SemaphoreType.DMA,
        ],
    )
    def gather_max(bb_hbm, idx_hbm, m_hbm, idx_v, acc_v, buf_v, sem):
        wid = lax.axis_index("s") * nc + lax.axis_index("c")
        for t in range(iters):
            ci = wid + t * nw

            @pl.when(ci < tot_chunks)
            def _():
                b = ci // chunks_per_b
                nb = (ci % chunks_per_b) * _CHUNK
                pltpu.sync_copy(idx_hbm.at[pl.ds(b * _K * _N + nb, _CHUNK)],
                                idx_v)
                pltpu.async_copy(bb_hbm.at[idx_v], acc_v, sem).wait()
                for k in range(1, _K):
                    pltpu.sync_copy(
                        idx_hbm.at[pl.ds((b * _K + k) * _N + nb, _CHUNK)],
                        idx_v)
                    pltpu.async_copy(bb_hbm.at[idx_v], buf_v, sem).wait()

                    def body(i, _):
                        for c in range(nvec):
                            sl = pl.ds(c * 16, 16)
                            acc_v[i, sl] = jnp.maximum(acc_v[i, sl],
                                                       buf_v[i, sl])
                        return 0

                    lax.fori_loop(0, _CHUNK, body, 0)
                pltpu.sync_copy(acc_v, m_hbm.at[pl.ds(b * _N + nb, _CHUNK)])

    return gather_max


def kernel(x, fc1_w, fc1_b, edge_w, edge_b, fc2_w, fc2_b):
    xf = x.reshape(_B, _C, _N)
    b1 = fc1_b.reshape(1, _C)
    w1m2 = edge_w[:, :_C] - edge_w[:, _C:]
    w2 = edge_w[:, _C:]
    be = edge_b.reshape(1, _COUT)
    b2 = fc2_b.reshape(_C, 1)

    xn, a, bb = pl.pallas_call(
        _pre_body,
        grid=(_B,),
        in_specs=[
            pl.BlockSpec((1, _C, _N), lambda b: (b, 0, 0)),
            pl.BlockSpec((_C, _C), lambda b: (0, 0)),
            pl.BlockSpec((1, _C), lambda b: (0, 0)),
            pl.BlockSpec((_COUT, _C), lambda b: (0, 0)),
            pl.BlockSpec((_COUT, _C), lambda b: (0, 0)),
            pl.BlockSpec((1, _COUT), lambda b: (0, 0)),
        ],
        out_specs=[
            pl.BlockSpec((1, _N, _C), lambda b: (b, 0, 0)),
            pl.BlockSpec((1, _N, _COUT), lambda b: (b, 0, 0)),
            pl.BlockSpec((1, _N, _CP), lambda b: (b, 0, 0)),
        ],
        out_shape=[
            jax.ShapeDtypeStruct((_B, _N, _C), jnp.float32),
            jax.ShapeDtypeStruct((_B, _N, _COUT), jnp.float32),
            jax.ShapeDtypeStruct((_B, _N, _CP), jnp.float32),
        ],
    )(xf, fc1_w, b1, w1m2, w2, be)

    gidx = pl.pallas_call(
        _knn_body,
        grid=(_B, _NBLK),
        in_specs=[
            pl.BlockSpec((1, _RB, _C), lambda b, j: (b, j, 0)),
            pl.BlockSpec((1, _N, _C), lambda b, j: (b, 0, 0)),
        ],
        out_specs=pl.BlockSpec((1, _RB, _K), lambda b, j: (b, j, 0)),
        out_shape=jax.ShapeDtypeStruct((_B, _N, _K), jnp.int32),
    )(xn, xn)

    idx_t = jnp.transpose(gidx, (0, 2, 1)).reshape(_B * _K * _N)
    m = _make_gather_max()(bb.reshape(_B * _N, _CP), idx_t)

    out = pl.pallas_call(
        _post_body,
        grid=(_B,),
        in_specs=[
            pl.BlockSpec((1, _N, _COUT), lambda b: (b, 0, 0)),
            pl.BlockSpec((1, _N, _CP), lambda b: (b, 0, 0)),
            pl.BlockSpec((_C, _COUT), lambda b: (0, 0)),
            pl.BlockSpec((_C, 1), lambda b: (0, 0)),
            pl.BlockSpec((1, _C, _N), lambda b: (b, 0, 0)),
        ],
        out_specs=pl.BlockSpec((1, _C, _N), lambda b: (b, 0, 0)),
        out_shape=jax.ShapeDtypeStruct((_B, _C, _N), jnp.float32),
    )(a, m.reshape(_B, _N, _CP), fc2_w, b2, xf)

    return out.reshape(_B, _C, _H, _W)


# speedup vs baseline: 636.2698x; 1.2405x over previous
import functools

import jax
import jax.numpy as jnp
from jax import lax
from jax.experimental import pallas as pl
from jax.experimental.pallas import tpu as pltpu
from jax.experimental.pallas import tpu_sc as plsc

_B, _C, _H, _W = 2, 96, 56, 56
_N = _H * _W
_K = 9
_COUT = 2 * _C
_RB = 392
_NBLK = _N // _RB
_CHUNK = 112
_CP = 256
_EPS = 1e-5


def _pre_body(x_ref, w1_ref, b1_ref, w1m2_ref, w2_ref, be_ref,
              xn_ref, a_ref, bb_ref):
    xb = x_ref[0]
    h = lax.dot_general(xb, w1_ref[...], (((0,), (1,)), ((), ())),
                        preferred_element_type=jnp.float32)
    h = h + b1_ref[...]
    mu = jnp.mean(h, axis=0, keepdims=True)
    var = jnp.mean((h - mu) ** 2, axis=0, keepdims=True)
    h = (h - mu) * lax.rsqrt(var + _EPS)
    nrm = jnp.sqrt(jnp.sum(h * h, axis=1, keepdims=True))
    xn_ref[0] = h / jnp.maximum(nrm, 1e-12)
    a_ref[0] = lax.dot_general(h, w1m2_ref[...], (((1,), (1,)), ((), ())),
                               preferred_element_type=jnp.float32) + be_ref[...]
    bbv = lax.dot_general(h, w2_ref[...], (((1,), (1,)), ((), ())),
                          preferred_element_type=jnp.float32)
    bb_ref[0] = jnp.concatenate(
        [bbv, jnp.zeros((_N, _CP - _COUT), jnp.float32)], axis=1)


def _knn_body(xb_ref, xa_ref, idx_ref):
    b = pl.program_id(0)
    xb = xb_ref[0]
    xa = xa_ref[0]
    d = lax.dot_general(xb, xa, (((1,), (1,)), ((), ())),
                        preferred_element_type=jnp.float32)
    cols = lax.broadcasted_iota(jnp.int32, (_RB, _N), 1)
    key = (d * 262144.0).astype(jnp.int32) * 4096 + (4095 - cols)
    picks = []
    for _ in range(_K):
        kv = jnp.max(key, axis=1, keepdims=True)
        picks.append((4095 - (kv & 4095)) + b * _N)
        key = jnp.where(key == kv, jnp.int32(-2**31), key)
    idx_ref[0] = jnp.concatenate(picks, axis=1)


def _post_body(a_ref, m_ref, w2_ref, b2_ref, x_ref, o_ref):
    y = jnp.maximum(a_ref[0] + m_ref[0][:, :_COUT], 0.0)
    z = lax.dot_general(w2_ref[...], y, (((1,), (1,)), ((), ())),
                        preferred_element_type=jnp.float32)
    z = z + b2_ref[...]
    mu = jnp.mean(z, axis=1, keepdims=True)
    var = jnp.mean((z - mu) ** 2, axis=1, keepdims=True)
    o_ref[0] = (z - mu) * lax.rsqrt(var + _EPS) + x_ref[0]


@functools.lru_cache(maxsize=1)
def _make_gather_max():
    info = plsc.get_sparse_core_info()
    nc, ns = info.num_cores, info.num_subcores
    nw = nc * ns
    tot_chunks = (_B * _N) // _CHUNK
    iters = -(-tot_chunks // nw)
    chunks_per_b = _N // _CHUNK
    nvec = _CP // 16

    mesh = plsc.VectorSubcoreMesh(core_axis_name="c", subcore_axis_name="s")

    @functools.partial(
        pl.kernel, mesh=mesh,
        out_type=jax.ShapeDtypeStruct((_B * _N, _CP), jnp.float32),
        scratch_types=[
            pltpu.VMEM((_CHUNK,), jnp.int32),
            pltpu.VMEM((_CHUNK, _CP), jnp.float32),
            pltpu.VMEM((_CHUNK, _CP), jnp.float32),
            pltpu.SemaphoreType.DMA,
        ],
    )
    def gather_max(bb_hbm, idx_hbm, m_hbm, idx_v, acc_v, buf_v, sem):
        wid = lax.axis_index("s") * nc + lax.axis_index("c")
        for t in range(iters):
            ci = wid + t * nw

            @pl.when(ci < tot_chunks)
            def _():
                b = ci // chunks_per_b
                nb = (ci % chunks_per_b) * _CHUNK
                pltpu.sync_copy(idx_hbm.at[pl.ds(b * _K * _N + nb, _CHUNK)],
                                idx_v)
                pltpu.async_copy(bb_hbm.at[idx_v], acc_v, sem).wait()
                for k in range(1, _K):
                    pltpu.sync_copy(
                        idx_hbm.at[pl.ds((b * _K + k) * _N + nb, _CHUNK)],
                        idx_v)
                    pltpu.async_copy(bb_hbm.at[idx_v], buf_v, sem).wait()

                    def body(i, _):
                        for c in range(nvec):
                            sl = pl.ds(c * 16, 16)
                            acc_v[i, sl] = jnp.maximum(acc_v[i, sl],
                                                       buf_v[i, sl])
                        return 0

                    lax.fori_loop(0, _CHUNK, body, 0)
                pltpu.sync_copy(acc_v, m_hbm.at[pl.ds(b * _N + nb, _CHUNK)])

    return gather_max


def kernel(x, fc1_w, fc1_b, edge_w, edge_b, fc2_w, fc2_b):
    xf = x.reshape(_B, _C, _N)
    b1 = fc1_b.reshape(1, _C)
    w1m2 = edge_w[:, :_C] - edge_w[:, _C:]
    w2 = edge_w[:, _C:]
    be = edge_b.reshape(1, _COUT)
    b2 = fc2_b.reshape(_C, 1)

    xn, a, bb = pl.pallas_call(
        _pre_body,
        grid=(_B,),
        in_specs=[
            pl.BlockSpec((1, _C, _N), lambda b: (b, 0, 0)),
            pl.BlockSpec((_C, _C), lambda b: (0, 0)),
            pl.BlockSpec((1, _C), lambda b: (0, 0)),
            pl.BlockSpec((_COUT, _C), lambda b: (0, 0)),
            pl.BlockSpec((_COUT, _C), lambda b: (0, 0)),
            pl.BlockSpec((1, _COUT), lambda b: (0, 0)),
        ],
        out_specs=[
            pl.BlockSpec((1, _N, _C), lambda b: (b, 0, 0)),
            pl.BlockSpec((1, _N, _COUT), lambda b: (b, 0, 0)),
            pl.BlockSpec((1, _N, _CP), lambda b: (b, 0, 0)),
        ],
        out_shape=[
            jax.ShapeDtypeStruct((_B, _N, _C), jnp.float32),
            jax.ShapeDtypeStruct((_B, _N, _COUT), jnp.float32),
            jax.ShapeDtypeStruct((_B, _N, _CP), jnp.float32),
        ],
    )(xf, fc1_w, b1, w1m2, w2, be)

    gidx = pl.pallas_call(
        _knn_body,
        grid=(_B, _NBLK),
        in_specs=[
            pl.BlockSpec((1, _RB, _C), lambda b, j: (b, j, 0)),
            pl.BlockSpec((1, _N, _C), lambda b, j: (b, 0, 0)),
        ],
        out_specs=pl.BlockSpec((1, _RB, _K), lambda b, j: (b, j, 0)),
        out_shape=jax.ShapeDtypeStruct((_B, _N, _K), jnp.int32),
    )(xn, xn)

    idx_t = jnp.transpose(gidx, (0, 2, 1)).reshape(_B * _K * _N)
    m = _make_gather_max()(bb.reshape(_B * _N, _CP), idx_t)

    out = pl.pallas_call(
        _post_body,
        grid=(_B,),
        in_specs=[
            pl.BlockSpec((1, _N, _COUT), lambda b: (b, 0, 0)),
            pl.BlockSpec((1, _N, _CP), lambda b: (b, 0, 0)),
            pl.BlockSpec((_C, _COUT), lambda b: (0, 0)),
            pl.BlockSpec((_C, 1), lambda b: (0, 0)),
            pl.BlockSpec((1, _C, _N), lambda b: (b, 0, 0)),
        ],
        out_specs=pl.BlockSpec((1, _C, _N), lambda b: (b, 0, 0)),
        out_shape=jax.ShapeDtypeStruct((_B, _C, _N), jnp.float32),
    )(a, m.reshape(_B, _N, _CP), fc2_w, b2, xf)

    return out.reshape(_B, _C, _H, _W)


# speedup vs baseline: 722.4098x; 1.1354x over previous
import functools

import jax
import jax.numpy as jnp
from jax import lax
from jax.experimental import pallas as pl
from jax.experimental.pallas import tpu as pltpu
from jax.experimental.pallas import tpu_sc as plsc

_B, _C, _H, _W = 2, 96, 56, 56
_N = _H * _W
_K = 9
_COUT = 2 * _C
_RB = 392
_NBLK = _N // _RB
_CHUNK = 112
_CP = 256
_EPS = 1e-5


def _pre_body(x_ref, w1_ref, b1_ref, w1m2_ref, w2_ref, be_ref,
              xn_ref, a_ref, bb_ref):
    xb = x_ref[0]
    h = lax.dot_general(xb, w1_ref[...], (((0,), (1,)), ((), ())),
                        preferred_element_type=jnp.float32)
    h = h + b1_ref[...]
    mu = jnp.mean(h, axis=0, keepdims=True)
    var = jnp.mean((h - mu) ** 2, axis=0, keepdims=True)
    h = (h - mu) * lax.rsqrt(var + _EPS)
    nrm = jnp.sqrt(jnp.sum(h * h, axis=1, keepdims=True))
    xn_ref[0] = h / jnp.maximum(nrm, 1e-12)
    a_ref[0] = lax.dot_general(h, w1m2_ref[...], (((1,), (1,)), ((), ())),
                               preferred_element_type=jnp.float32) + be_ref[...]
    bbv = lax.dot_general(h, w2_ref[...], (((1,), (1,)), ((), ())),
                          preferred_element_type=jnp.float32)
    bb_ref[0] = jnp.concatenate(
        [bbv, jnp.zeros((_N, _CP - _COUT), jnp.float32)], axis=1)


def _knn_body(xb_ref, xa_ref, idx_ref):
    b = pl.program_id(0)
    xb = xb_ref[0]
    xa = xa_ref[0]
    d = lax.dot_general(xb, xa, (((1,), (1,)), ((), ())),
                        preferred_element_type=jnp.float32)
    cols = lax.broadcasted_iota(jnp.int32, (_RB, _N), 1)
    key = (d * 262144.0).astype(jnp.int32) * 4096 + (4095 - cols)
    kv = jnp.max(key, axis=1, keepdims=True)
    picks = [(4095 - (kv & 4095)) + b * _N]
    for _ in range(1, _K):
        kv = jnp.max(jnp.where(key < kv, key, jnp.int32(-2**31)),
                     axis=1, keepdims=True)
        picks.append((4095 - (kv & 4095)) + b * _N)
    idx_ref[0] = jnp.concatenate(picks, axis=1)


def _post_body(a_ref, m_ref, w2_ref, b2_ref, x_ref, o_ref):
    y = jnp.maximum(a_ref[0] + m_ref[0][:, :_COUT], 0.0)
    z = lax.dot_general(w2_ref[...], y, (((1,), (1,)), ((), ())),
                        preferred_element_type=jnp.float32)
    z = z + b2_ref[...]
    mu = jnp.mean(z, axis=1, keepdims=True)
    var = jnp.mean((z - mu) ** 2, axis=1, keepdims=True)
    o_ref[0] = (z - mu) * lax.rsqrt(var + _EPS) + x_ref[0]


@functools.lru_cache(maxsize=1)
def _make_gather_max():
    info = plsc.get_sparse_core_info()
    nc, ns = info.num_cores, info.num_subcores
    nw = nc * ns
    tot_chunks = (_B * _N) // _CHUNK
    iters = -(-tot_chunks // nw)
    chunks_per_b = _N // _CHUNK
    nvec = _CP // 16

    mesh = plsc.VectorSubcoreMesh(core_axis_name="c", subcore_axis_name="s")

    @functools.partial(
        pl.kernel, mesh=mesh,
        out_type=jax.ShapeDtypeStruct((_B * _N, _CP), jnp.float32),
        scratch_types=[
            pltpu.VMEM((_CHUNK,), jnp.int32),
            pltpu.VMEM((_CHUNK,), jnp.int32),
            pltpu.VMEM((_CHUNK, _CP), jnp.float32),
            pltpu.VMEM((_CHUNK, _CP), jnp.float32),
            pltpu.VMEM((_CHUNK, _CP), jnp.float32),
            pltpu.SemaphoreType.DMA,
            pltpu.SemaphoreType.DMA,
            pltpu.SemaphoreType.DMA,
        ],
    )
    def gather_max(bb_hbm, idx_hbm, m_hbm, idx0_v, idx1_v, acc_v,
                   buf0_v, buf1_v, sema, sem0, sem1):
        wid = lax.axis_index("s") * nc + lax.axis_index("c")
        idx = (idx0_v, idx1_v)
        buf = (buf0_v, buf1_v)
        sem = (sem0, sem1)
        for t in range(iters):
            ci = wid + t * nw

            @pl.when(ci < tot_chunks)
            def _():
                b = ci // chunks_per_b
                nb = (ci % chunks_per_b) * _CHUNK

                def idx_src(k):
                    return idx_hbm.at[pl.ds((b * _K + k) * _N + nb, _CHUNK)]

                pltpu.sync_copy(idx_src(0), idx[0])
                ha = pltpu.async_copy(bb_hbm.at[idx[0]], acc_v, sema)
                pltpu.sync_copy(idx_src(1), idx[1])
                h = [pltpu.async_copy(bb_hbm.at[idx[1]], buf[0], sem[0]),
                     None]
                ha.wait()
                for k in range(1, _K):
                    nxt = k + 1
                    if nxt < _K:
                        s = (nxt - 1) % 2
                        pltpu.sync_copy(idx_src(nxt), idx[nxt % 2])
                        h[s] = pltpu.async_copy(bb_hbm.at[idx[nxt % 2]],
                                                buf[s], sem[s])
                    cur = (k - 1) % 2
                    h[cur].wait()
                    src_v = buf[cur]

                    def body(i, _):
                        for c in range(nvec):
                            sl = pl.ds(c * 16, 16)
                            acc_v[i, sl] = jnp.maximum(acc_v[i, sl],
                                                       src_v[i, sl])
                        return 0

                    lax.fori_loop(0, _CHUNK, body, 0)
                pltpu.sync_copy(acc_v, m_hbm.at[pl.ds(b * _N + nb, _CHUNK)])

    return gather_max


def kernel(x, fc1_w, fc1_b, edge_w, edge_b, fc2_w, fc2_b):
    xf = x.reshape(_B, _C, _N)
    b1 = fc1_b.reshape(1, _C)
    w1m2 = edge_w[:, :_C] - edge_w[:, _C:]
    w2 = edge_w[:, _C:]
    be = edge_b.reshape(1, _COUT)
    b2 = fc2_b.reshape(_C, 1)

    xn, a, bb = pl.pallas_call(
        _pre_body,
        grid=(_B,),
        in_specs=[
            pl.BlockSpec((1, _C, _N), lambda b: (b, 0, 0)),
            pl.BlockSpec((_C, _C), lambda b: (0, 0)),
            pl.BlockSpec((1, _C), lambda b: (0, 0)),
            pl.BlockSpec((_COUT, _C), lambda b: (0, 0)),
            pl.BlockSpec((_COUT, _C), lambda b: (0, 0)),
            pl.BlockSpec((1, _COUT), lambda b: (0, 0)),
        ],
        out_specs=[
            pl.BlockSpec((1, _N, _C), lambda b: (b, 0, 0)),
            pl.BlockSpec((1, _N, _COUT), lambda b: (b, 0, 0)),
            pl.BlockSpec((1, _N, _CP), lambda b: (b, 0, 0)),
        ],
        out_shape=[
            jax.ShapeDtypeStruct((_B, _N, _C), jnp.float32),
            jax.ShapeDtypeStruct((_B, _N, _COUT), jnp.float32),
            jax.ShapeDtypeStruct((_B, _N, _CP), jnp.float32),
        ],
    )(xf, fc1_w, b1, w1m2, w2, be)

    gidx = pl.pallas_call(
        _knn_body,
        grid=(_B, _NBLK),
        in_specs=[
            pl.BlockSpec((1, _RB, _C), lambda b, j: (b, j, 0)),
            pl.BlockSpec((1, _N, _C), lambda b, j: (b, 0, 0)),
        ],
        out_specs=pl.BlockSpec((1, _RB, _K), lambda b, j: (b, j, 0)),
        out_shape=jax.ShapeDtypeStruct((_B, _N, _K), jnp.int32),
    )(xn, xn)

    idx_t = jnp.transpose(gidx, (0, 2, 1)).reshape(_B * _K * _N)
    m = _make_gather_max()(bb.reshape(_B * _N, _CP), idx_t)

    out = pl.pallas_call(
        _post_body,
        grid=(_B,),
        in_specs=[
            pl.BlockSpec((1, _N, _COUT), lambda b: (b, 0, 0)),
            pl.BlockSpec((1, _N, _CP), lambda b: (b, 0, 0)),
            pl.BlockSpec((_C, _COUT), lambda b: (0, 0)),
            pl.BlockSpec((_C, 1), lambda b: (0, 0)),
            pl.BlockSpec((1, _C, _N), lambda b: (b, 0, 0)),
        ],
        out_specs=pl.BlockSpec((1, _C, _N), lambda b: (b, 0, 0)),
        out_shape=jax.ShapeDtypeStruct((_B, _C, _N), jnp.float32),
    )(a, m.reshape(_B, _N, _CP), fc2_w, b2, xf)

    return out.reshape(_B, _C, _H, _W)


# speedup vs baseline: 802.2456x; 1.1105x over previous
import functools

import jax
import jax.numpy as jnp
from jax import lax
from jax.experimental import pallas as pl
from jax.experimental.pallas import tpu as pltpu
from jax.experimental.pallas import tpu_sc as plsc

_B, _C, _H, _W = 2, 96, 56, 56
_N = _H * _W
_K = 9
_COUT = 2 * _C
_RB = 392
_NBLK = _N // _RB
_CHUNK = 112
_CP = 256
_EPS = 1e-5


def _pre_body(x_ref, w1_ref, b1_ref, w1m2_ref, w2_ref, be_ref,
              xn0_ref, xn1_ref, a0_ref, a1_ref, bb0_ref, bb1_ref):
    for b, (xn_ref, a_ref, bb_ref) in enumerate(
            ((xn0_ref, a0_ref, bb0_ref), (xn1_ref, a1_ref, bb1_ref))):
        xb = x_ref[b]
        h = lax.dot_general(xb, w1_ref[...], (((0,), (1,)), ((), ())),
                            preferred_element_type=jnp.float32)
        h = h + b1_ref[...]
        mu = jnp.mean(h, axis=0, keepdims=True)
        var = jnp.mean((h - mu) ** 2, axis=0, keepdims=True)
        h = (h - mu) * lax.rsqrt(var + _EPS)
        nrm = jnp.sqrt(jnp.sum(h * h, axis=1, keepdims=True))
        xn_ref[...] = h / jnp.maximum(nrm, 1e-12)
        a_ref[...] = lax.dot_general(
            h, w1m2_ref[...], (((1,), (1,)), ((), ())),
            preferred_element_type=jnp.float32) + be_ref[...]
        bbv = lax.dot_general(h, w2_ref[...], (((1,), (1,)), ((), ())),
                              preferred_element_type=jnp.float32)
        bb_ref[...] = jnp.concatenate(
            [bbv, jnp.zeros((_N, _CP - _COUT), jnp.float32)], axis=1)


def _knn_body(xb_ref, xa_ref, idx_ref):
    xb = xb_ref[...]
    xa = xa_ref[...]
    d = lax.dot_general(xb, xa, (((1,), (1,)), ((), ())),
                        preferred_element_type=jnp.float32)
    cols = lax.broadcasted_iota(jnp.int32, (_RB, _N), 1)
    key = (d * 262144.0).astype(jnp.int32) * 4096 + (4095 - cols)
    kv = jnp.max(key, axis=1, keepdims=True)
    picks = [4095 - (kv & 4095)]
    for _ in range(1, _K):
        kv = jnp.max(jnp.where(key < kv, key, jnp.int32(-2**31)),
                     axis=1, keepdims=True)
        picks.append(4095 - (kv & 4095))
    idx_ref[...] = jnp.concatenate(picks, axis=1)


def _post_body(a_ref, m_ref, w2_ref, b2_ref, x_ref, o_ref):
    y = jnp.maximum(a_ref[...] + m_ref[..., :_COUT], 0.0)
    z = lax.dot_general(w2_ref[...], y, (((1,), (1,)), ((), ())),
                        preferred_element_type=jnp.float32)
    z = z + b2_ref[...]
    mu = jnp.mean(z, axis=1, keepdims=True)
    var = jnp.mean((z - mu) ** 2, axis=1, keepdims=True)
    o_ref[...] = (z - mu) * lax.rsqrt(var + _EPS) + x_ref[...]


@functools.lru_cache(maxsize=1)
def _make_gather_max():
    info = plsc.get_sparse_core_info()
    nc, ns = info.num_cores, info.num_subcores
    tot_chunks = _N // _CHUNK
    nvec = _CP // 16

    mesh = plsc.VectorSubcoreMesh(core_axis_name="c", subcore_axis_name="s")

    @functools.partial(
        pl.kernel, mesh=mesh,
        out_type=jax.ShapeDtypeStruct((_N, _CP), jnp.float32),
        scratch_types=[
            pltpu.VMEM((_CHUNK,), jnp.int32),
            pltpu.VMEM((_CHUNK,), jnp.int32),
            pltpu.VMEM((_CHUNK, _CP), jnp.float32),
            pltpu.VMEM((_CHUNK, _CP), jnp.float32),
            pltpu.VMEM((_CHUNK, _CP), jnp.float32),
            pltpu.SemaphoreType.DMA,
            pltpu.SemaphoreType.DMA,
            pltpu.SemaphoreType.DMA,
        ],
    )
    def gather_max(bb_hbm, idx_hbm, m_hbm, idx0_v, idx1_v, acc_v,
                   buf0_v, buf1_v, sema, sem0, sem1):
        ci = lax.axis_index("s") * nc + lax.axis_index("c")
        idx = (idx0_v, idx1_v)
        buf = (buf0_v, buf1_v)
        sem = (sem0, sem1)

        @pl.when(ci < tot_chunks)
        def _():
            nb = ci * _CHUNK

            def idx_src(k):
                return idx_hbm.at[pl.ds(k * _N + nb, _CHUNK)]

            pltpu.sync_copy(idx_src(0), idx[0])
            ha = pltpu.async_copy(bb_hbm.at[idx[0]], acc_v, sema)
            pltpu.sync_copy(idx_src(1), idx[1])
            h = [pltpu.async_copy(bb_hbm.at[idx[1]], buf[0], sem[0]),
                 None]
            ha.wait()
            for k in range(1, _K):
                nxt = k + 1
                if nxt < _K:
                    s = (nxt - 1) % 2
                    pltpu.sync_copy(idx_src(nxt), idx[nxt % 2])
                    h[s] = pltpu.async_copy(bb_hbm.at[idx[nxt % 2]],
                                            buf[s], sem[s])
                cur = (k - 1) % 2
                h[cur].wait()
                src_v = buf[cur]

                def body(i, _):
                    for c in range(nvec):
                        sl = pl.ds(c * 16, 16)
                        acc_v[i, sl] = jnp.maximum(acc_v[i, sl],
                                                   src_v[i, sl])
                    return 0

                lax.fori_loop(0, _CHUNK, body, 0)
            pltpu.sync_copy(acc_v, m_hbm.at[pl.ds(nb, _CHUNK)])

    return gather_max


_full2 = lambda: (0, 0)


def kernel(x, fc1_w, fc1_b, edge_w, edge_b, fc2_w, fc2_b):
    xf = x.reshape(_B, _C, _N)
    b1 = fc1_b.reshape(1, _C)
    w1m2 = edge_w[:, :_C] - edge_w[:, _C:]
    w2 = edge_w[:, _C:]
    be = edge_b.reshape(1, _COUT)
    b2 = fc2_b.reshape(_C, 1)

    pre_out = pl.pallas_call(
        _pre_body,
        in_specs=[
            pl.BlockSpec((_B, _C, _N), lambda: (0, 0, 0)),
            pl.BlockSpec((_C, _C), lambda: (0, 0)),
            pl.BlockSpec((1, _C), lambda: (0, 0)),
            pl.BlockSpec((_COUT, _C), lambda: (0, 0)),
            pl.BlockSpec((_COUT, _C), lambda: (0, 0)),
            pl.BlockSpec((1, _COUT), lambda: (0, 0)),
        ],
        out_shape=[
            jax.ShapeDtypeStruct((_N, _C), jnp.float32),
            jax.ShapeDtypeStruct((_N, _C), jnp.float32),
            jax.ShapeDtypeStruct((_N, _COUT), jnp.float32),
            jax.ShapeDtypeStruct((_N, _COUT), jnp.float32),
            jax.ShapeDtypeStruct((_N, _CP), jnp.float32),
            jax.ShapeDtypeStruct((_N, _CP), jnp.float32),
        ],
    )(xf, fc1_w, b1, w1m2, w2, be)
    xn = pre_out[0:2]
    a = pre_out[2:4]
    bb = pre_out[4:6]

    knn = pl.pallas_call(
        _knn_body,
        grid=(_NBLK,),
        in_specs=[
            pl.BlockSpec((_RB, _C), lambda j: (j, 0)),
            pl.BlockSpec((_N, _C), lambda j: (0, 0)),
        ],
        out_specs=pl.BlockSpec((_RB, _K), lambda j: (j, 0)),
        out_shape=jax.ShapeDtypeStruct((_N, _K), jnp.int32),
    )

    post = pl.pallas_call(
        _post_body,
        in_specs=[
            pl.BlockSpec((_N, _COUT), _full2),
            pl.BlockSpec((_N, _CP), _full2),
            pl.BlockSpec((_C, _COUT), _full2),
            pl.BlockSpec((_C, 1), _full2),
            pl.BlockSpec((_C, _N), _full2),
        ],
        out_shape=jax.ShapeDtypeStruct((_C, _N), jnp.float32),
    )

    gather_max = _make_gather_max()
    outs = []
    for b in range(_B):
        gidx = knn(xn[b], xn[b])
        idx_t = jnp.transpose(gidx, (1, 0)).reshape(_K * _N)
        m = gather_max(bb[b], idx_t)
        outs.append(post(a[b], m, fc2_w, b2, xf[b]))

    return jnp.stack(outs, axis=0).reshape(_B, _C, _H, _W)


# speedup vs baseline: 804.9994x; 1.0034x over previous
import functools

import jax
import jax.numpy as jnp
from jax import lax
from jax.experimental import pallas as pl
from jax.experimental.pallas import tpu as pltpu
from jax.experimental.pallas import tpu_sc as plsc

_B, _C, _H, _W = 2, 96, 56, 56
_N = _H * _W
_NH = _N // 2
_K = 9
_COUT = 2 * _C
_RB = 392
_NBLK = _NH // _RB
_CHUNK = 56
_CP = 256
_EPS = 1e-5


def _pre_body(x_ref, w1_ref, b1_ref, w1m2_ref, w2_ref, be_ref,
              xn0_ref, xn1_ref, a0_ref, a1_ref, bb0_ref, bb1_ref):
    for b, (xn_ref, a_ref, bb_ref) in enumerate(
            ((xn0_ref, a0_ref, bb0_ref), (xn1_ref, a1_ref, bb1_ref))):
        xb = x_ref[b]
        h = lax.dot_general(xb, w1_ref[...], (((0,), (1,)), ((), ())),
                            preferred_element_type=jnp.float32)
        h = h + b1_ref[...]
        mu = jnp.mean(h, axis=0, keepdims=True)
        var = jnp.mean((h - mu) ** 2, axis=0, keepdims=True)
        h = (h - mu) * lax.rsqrt(var + _EPS)
        nrm = jnp.sqrt(jnp.sum(h * h, axis=1, keepdims=True))
        xn_ref[...] = h / jnp.maximum(nrm, 1e-12)
        a_ref[...] = lax.dot_general(
            h, w1m2_ref[...], (((1,), (1,)), ((), ())),
            preferred_element_type=jnp.float32) + be_ref[...]
        bbv = lax.dot_general(h, w2_ref[...], (((1,), (1,)), ((), ())),
                              preferred_element_type=jnp.float32)
        bb_ref[...] = jnp.concatenate(
            [bbv, jnp.zeros((_N, _CP - _COUT), jnp.float32)], axis=1)


def _knn_body(xb_ref, xa_ref, idx_ref):
    xb = xb_ref[...]
    xa = xa_ref[...]
    d = lax.dot_general(xb, xa, (((1,), (1,)), ((), ())),
                        preferred_element_type=jnp.float32)
    cols = lax.broadcasted_iota(jnp.int32, (_RB, _N), 1)
    key = (d * 262144.0).astype(jnp.int32) * 4096 + (4095 - cols)
    kv = jnp.max(key, axis=1, keepdims=True)
    picks = [4095 - (kv & 4095)]
    for _ in range(1, _K):
        kv = jnp.max(jnp.where(key < kv, key, jnp.int32(-2**31)),
                     axis=1, keepdims=True)
        picks.append(4095 - (kv & 4095))
    idx_ref[...] = jnp.concatenate(picks, axis=1)


def _post_body(a_ref, m0_ref, m1_ref, w2_ref, b2_ref, x_ref, o_ref):
    m = jnp.concatenate([m0_ref[..., :_COUT], m1_ref[..., :_COUT]], axis=0)
    y = jnp.maximum(a_ref[...] + m, 0.0)
    z = lax.dot_general(w2_ref[...], y, (((1,), (1,)), ((), ())),
                        preferred_element_type=jnp.float32)
    z = z + b2_ref[...]
    mu = jnp.mean(z, axis=1, keepdims=True)
    var = jnp.mean((z - mu) ** 2, axis=1, keepdims=True)
    o_ref[...] = (z - mu) * lax.rsqrt(var + _EPS) + x_ref[0]


@functools.lru_cache(maxsize=1)
def _make_gather_max():
    info = plsc.get_sparse_core_info()
    nc, ns = info.num_cores, info.num_subcores
    tot_chunks = _NH // _CHUNK
    nvec = _CP // 16

    mesh = plsc.VectorSubcoreMesh(core_axis_name="c", subcore_axis_name="s")

    @functools.partial(
        pl.kernel, mesh=mesh,
        out_type=jax.ShapeDtypeStruct((_NH, _CP), jnp.float32),
        scratch_types=[
            pltpu.VMEM((_CHUNK,), jnp.int32),
            pltpu.VMEM((_CHUNK,), jnp.int32),
            pltpu.VMEM((_CHUNK, _CP), jnp.float32),
            pltpu.VMEM((_CHUNK, _CP), jnp.float32),
            pltpu.VMEM((_CHUNK, _CP), jnp.float32),
            pltpu.SemaphoreType.DMA,
            pltpu.SemaphoreType.DMA,
            pltpu.SemaphoreType.DMA,
        ],
    )
    def gather_max(bb_hbm, idx_hbm, m_hbm, idx0_v, idx1_v, acc_v,
                   buf0_v, buf1_v, sema, sem0, sem1):
        ci = lax.axis_index("s") * nc + lax.axis_index("c")
        idx = (idx0_v, idx1_v)
        buf = (buf0_v, buf1_v)
        sem = (sem0, sem1)

        @pl.when(ci < tot_chunks)
        def _():
            nb = ci * _CHUNK

            def idx_src(k):
                return idx_hbm.at[pl.ds(k * _NH + nb, _CHUNK)]

            pltpu.sync_copy(idx_src(0), idx[0])
            ha = pltpu.async_copy(bb_hbm.at[idx[0]], acc_v, sema)
            pltpu.sync_copy(idx_src(1), idx[1])
            h = [pltpu.async_copy(bb_hbm.at[idx[1]], buf[0], sem[0]),
                 None]
            ha.wait()
            for k in range(1, _K):
                nxt = k + 1
                if nxt < _K:
                    s = (nxt - 1) % 2
                    pltpu.sync_copy(idx_src(nxt), idx[nxt % 2])
                    h[s] = pltpu.async_copy(bb_hbm.at[idx[nxt % 2]],
                                            buf[s], sem[s])
                cur = (k - 1) % 2
                h[cur].wait()
                src_v = buf[cur]

                def body(i, _):
                    for c in range(nvec):
                        sl = pl.ds(c * 16, 16)
                        acc_v[i, sl] = jnp.maximum(acc_v[i, sl],
                                                   src_v[i, sl])
                    return 0

                lax.fori_loop(0, _CHUNK, body, 0)
            pltpu.sync_copy(acc_v, m_hbm.at[pl.ds(nb, _CHUNK)])

    return gather_max


_full2 = lambda: (0, 0)


def kernel(x, fc1_w, fc1_b, edge_w, edge_b, fc2_w, fc2_b):
    xf = x.reshape(_B, _C, _N)
    b1 = fc1_b.reshape(1, _C)
    w1m2 = edge_w[:, :_C] - edge_w[:, _C:]
    w2 = edge_w[:, _C:]
    be = edge_b.reshape(1, _COUT)
    b2 = fc2_b.reshape(_C, 1)

    pre_out = pl.pallas_call(
        _pre_body,
        in_specs=[
            pl.BlockSpec((_B, _C, _N), lambda: (0, 0, 0)),
            pl.BlockSpec((_C, _C), _full2),
            pl.BlockSpec((1, _C), _full2),
            pl.BlockSpec((_COUT, _C), _full2),
            pl.BlockSpec((_COUT, _C), _full2),
            pl.BlockSpec((1, _COUT), _full2),
        ],
        out_shape=[
            jax.ShapeDtypeStruct((_N, _C), jnp.float32),
            jax.ShapeDtypeStruct((_N, _C), jnp.float32),
            jax.ShapeDtypeStruct((_N, _COUT), jnp.float32),
            jax.ShapeDtypeStruct((_N, _COUT), jnp.float32),
            jax.ShapeDtypeStruct((_N, _CP), jnp.float32),
            jax.ShapeDtypeStruct((_N, _CP), jnp.float32),
        ],
    )(xf, fc1_w, b1, w1m2, w2, be)
    xn = pre_out[0:2]
    a = pre_out[2:4]
    bb = pre_out[4:6]

    def make_knn(half):
        return pl.pallas_call(
            _knn_body,
            grid=(_NBLK,),
            in_specs=[
                pl.BlockSpec((_RB, _C), lambda j: (j + half * _NBLK, 0)),
                pl.BlockSpec((_N, _C), lambda j: (0, 0)),
            ],
            out_specs=pl.BlockSpec((_RB, _K), lambda j: (j, 0)),
            out_shape=jax.ShapeDtypeStruct((_NH, _K), jnp.int32),
        )

    knn = [make_knn(0), make_knn(1)]

    def make_post(b):
        f2 = lambda i: (0, 0)
        return pl.pallas_call(
            _post_body,
            grid=(1,),
            in_specs=[
                pl.BlockSpec((_N, _COUT), f2),
                pl.BlockSpec((_NH, _CP), f2),
                pl.BlockSpec((_NH, _CP), f2),
                pl.BlockSpec((_C, _COUT), f2),
                pl.BlockSpec((_C, 1), f2),
                pl.BlockSpec((1, _C, _N), lambda i: (b, 0, 0)),
            ],
            out_specs=pl.BlockSpec((_C, _N), f2),
            out_shape=jax.ShapeDtypeStruct((_C, _N), jnp.float32),
        )

    gather_max = _make_gather_max()
    m = [[None, None], [None, None]]
    for b in range(_B):
        for half in range(2):
            gidx = knn[half](xn[b], xn[b])
            idx_t = jnp.transpose(gidx, (1, 0)).reshape(_K * _NH)
            m[b][half] = gather_max(bb[b], idx_t)

    outs = [make_post(b)(a[b], m[b][0], m[b][1], fc2_w, b2, xf)
            for b in range(_B)]
    return jnp.stack(outs, axis=0).reshape(_B, _C, _H, _W)
